# baseline (speedup 1.0000x reference)
import functools

import jax
import jax.numpy as jnp
from jax import lax
from jax.experimental import pallas as pl
from jax.experimental.pallas import tpu as pltpu
from jax.experimental.pallas import tpu_sc as plsc

N = 10000
E = 320000
D = 128

NC = 2
NS = 16
NW = NC * NS
EPW = E // NW
CHUNK = 125
NCH = EPW // CHUNK
GRP = 20
CH3 = 50
NCH3 = EPW // CH3
NB = 20
NBLK = NCH3 // NB
DEPTH = 5
NP1 = 10240
DPT = NP1 // NS
LAST = N - (NS - 1) * DPT
RB = 2000


@functools.cache
def _sc_kernels():
    mesh = plsc.VectorSubcoreMesh(core_axis_name="c", subcore_axis_name="s")

    @functools.partial(
        pl.kernel,
        mesh=mesh,
        out_type=jax.ShapeDtypeStruct((NC, NP1), jnp.float32),
        scratch_types=[
            pltpu.VMEM((NCH, CHUNK), jnp.int32),
            pltpu.VMEM((128,), jnp.float32),
            pltpu.VMEM_SHARED((NP1,), jnp.float32),
            pltpu.SemaphoreType.DMA,
        ],
    )
    def deg_kernel(dst_hbm, z1_hbm, pdeg_hbm, dstv, onesv, deg, semd):
        c = lax.axis_index("c")
        s = lax.axis_index("s")
        wid = s * NC + c
        pltpu.sync_copy(z1_hbm.at[pl.ds(s * DPT, DPT)],
                        deg.at[pl.ds(s * DPT, DPT)])
        pltpu.sync_copy(dst_hbm.at[wid], dstv)
        one16 = jnp.ones((16,), jnp.float32)
        for i in range(128 // 16):
            onesv[pl.ds(i * 16, 16)] = one16
        plsc.subcore_barrier()

        def group(gi, carry):
            for i in range(GRP):
                pltpu.async_copy(onesv.at[pl.ds(0, CHUNK)],
                                 deg.at[dstv.at[gi * GRP + i]], semd,
                                 add=True)
            for i in range(GRP):
                pltpu.make_async_copy(onesv.at[pl.ds(0, CHUNK)],
                                      deg.at[dstv.at[gi * GRP + i]],
                                      semd).wait()
            return carry

        lax.fori_loop(0, NCH // GRP, group, 0)
        plsc.subcore_barrier()
        pltpu.sync_copy(deg.at[pl.ds(s * DPT, DPT)],
                        pdeg_hbm.at[c, pl.ds(s * DPT, DPT)])

    @functools.partial(
        pl.kernel,
        mesh=mesh,
        out_type=jax.ShapeDtypeStruct((NC, N, D), jnp.float32),
        scratch_types=[
            pltpu.VMEM((2, NB, CH3), jnp.int32),
            pltpu.VMEM((2, NB, CH3), jnp.int32),
            pltpu.VMEM((CH3, D), jnp.float32),
            pltpu.VMEM((CH3, D), jnp.float32),
            pltpu.VMEM((CH3, D), jnp.float32),
            pltpu.VMEM((CH3, D), jnp.float32),
            pltpu.VMEM((CH3, D), jnp.float32),
            pltpu.VMEM_SHARED((NP1, D), jnp.float32),
            pltpu.SemaphoreType.DMA,
            pltpu.SemaphoreType.DMA,
            pltpu.SemaphoreType.DMA,
            pltpu.SemaphoreType.DMA,
            pltpu.SemaphoreType.DMA,
            pltpu.SemaphoreType.DMA,
        ],
    )
    def scatter_kernel(g_hbm, src_hbm, dst_hbm, z2_hbm, pacc_hbm,
                       srcv, dstv, rows0, rows1, rows2, rows3, rows4, acc,
                       sem0, sem1, sem2, sem3, sem4, semi):
        c = lax.axis_index("c")
        s = lax.axis_index("s")
        wid = s * NC + c
        base = s * DPT

        @pl.when(jnp.logical_and(c == 0, s < NS - 1))
        def _():
            pltpu.sync_copy(g_hbm.at[pl.ds(base, DPT)],
                            acc.at[pl.ds(base, DPT)])

        @pl.when(jnp.logical_and(c == 0, s == NS - 1))
        def _():
            pltpu.sync_copy(g_hbm.at[pl.ds(base, LAST)],
                            acc.at[pl.ds(base, LAST)])

        @pl.when(c == 1)
        def _():
            pltpu.sync_copy(z2_hbm, acc.at[pl.ds(base, DPT)])

        pltpu.sync_copy(src_hbm.at[wid, 0], srcv.at[0])
        pltpu.sync_copy(dst_hbm.at[wid, 0], dstv.at[0])
        rbufs = ((rows0, sem0), (rows1, sem1), (rows2, sem2),
                 (rows3, sem3), (rows4, sem4))
        for k in range(DEPTH - 1):
            pltpu.async_copy(g_hbm.at[srcv.at[0, k]], *rbufs[k])
        plsc.subcore_barrier()

        def block_body(b, carry):
            pb = b % 2

            @pl.when(b + 1 < NBLK)
            def _():
                pltpu.async_copy(src_hbm.at[wid, b + 1], srcv.at[1 - pb],
                                 semi)
                pltpu.async_copy(dst_hbm.at[wid, b + 1], dstv.at[1 - pb],
                                 semi)

            for i in range(NB):
                cur, csem = rbufs[i % DEPTH]
                nxt, nsem = rbufs[(i + DEPTH - 1) % DEPTH]
                if i + DEPTH - 1 < NB:
                    pltpu.async_copy(g_hbm.at[srcv.at[pb, i + DEPTH - 1]],
                                     nxt, nsem)
                else:
                    if i + DEPTH - 1 == NB:
                        @pl.when(b + 1 < NBLK)
                        def _():
                            pltpu.make_async_copy(src_hbm.at[wid, b + 1],
                                                  srcv.at[1 - pb],
                                                  semi).wait()
                            pltpu.make_async_copy(dst_hbm.at[wid, b + 1],
                                                  dstv.at[1 - pb],
                                                  semi).wait()

                    @pl.when(b + 1 < NBLK)
                    def _():
                        pltpu.async_copy(
                            g_hbm.at[srcv.at[1 - pb, i + DEPTH - 1 - NB]],
                            nxt, nsem)
                pltpu.make_async_copy(g_hbm.at[srcv.at[pb, i]], cur,
                                      csem).wait()
                pltpu.sync_copy(cur, acc.at[dstv.at[pb, i]], add=True)
            return carry

        lax.fori_loop(0, NBLK, block_body, 0)
        plsc.subcore_barrier()

        @pl.when(s < NS - 1)
        def _():
            pltpu.sync_copy(acc.at[pl.ds(base, DPT)],
                            pacc_hbm.at[c, pl.ds(base, DPT)])

        @pl.when(s == NS - 1)
        def _():
            pltpu.sync_copy(acc.at[pl.ds(base, LAST)],
                            pacc_hbm.at[c, pl.ds(base, LAST)])

    return deg_kernel, scatter_kernel


def _scale_body(x_ref, w_ref, pdegt_ref, g_ref):
    deg = jnp.sum(pdegt_ref[...], axis=1, keepdims=True) + 1.0
    dinv = lax.rsqrt(deg)
    h = jnp.dot(x_ref[...], w_ref[...], preferred_element_type=jnp.float32)
    g_ref[...] = h * dinv


_scale_call = pl.pallas_call(
    _scale_body,
    grid=(N // RB,),
    in_specs=[
        pl.BlockSpec((RB, D), lambda i: (i, 0)),
        pl.BlockSpec((D, D), lambda i: (0, 0)),
        pl.BlockSpec((RB, NC), lambda i: (i, 0)),
    ],
    out_specs=pl.BlockSpec((RB, D), lambda i: (i, 0)),
    out_shape=jax.ShapeDtypeStruct((N, D), jnp.float32),
)


def _final_body(x_ref, pacc_ref, pdegt_ref, b_ref, o_ref):
    deg = jnp.sum(pdegt_ref[...], axis=1, keepdims=True) + 1.0
    dinv = lax.rsqrt(deg)
    tot = pacc_ref[0] + pacc_ref[1]
    o_ref[...] = x_ref[...] + jnp.maximum(tot * dinv + b_ref[...], 0.0)


_final_call = pl.pallas_call(
    _final_body,
    grid=(N // RB,),
    in_specs=[
        pl.BlockSpec((RB, D), lambda i: (i, 0)),
        pl.BlockSpec((NC, RB, D), lambda i: (0, i, 0)),
        pl.BlockSpec((RB, NC), lambda i: (i, 0)),
        pl.BlockSpec((1, D), lambda i: (0, 0)),
    ],
    out_specs=pl.BlockSpec((RB, D), lambda i: (i, 0)),
    out_shape=jax.ShapeDtypeStruct((N, D), jnp.float32),
)


def kernel(x, edge_index, W, b):
    ei = edge_index.astype(jnp.int32)
    srcb = ei[0].reshape(NW, NBLK, NB, CH3)
    dstb = ei[1].reshape(NW, NBLK, NB, CH3)
    dstk1 = ei[1].reshape(NW, NCH, CHUNK)
    z1 = jnp.zeros((NP1,), jnp.float32)
    z2 = jnp.zeros((DPT, D), jnp.float32)
    deg_kernel, scatter_kernel = _sc_kernels()
    pdeg = deg_kernel(dstk1, z1)
    pdegt = pdeg.T[:N]
    g = _scale_call(x, W, pdegt)
    pacc = scatter_kernel(g, srcb, dstb, z2)
    return _final_call(x, pacc, pdegt, b.reshape(1, D))
